# baseline (speedup 1.0000x reference)
import numpy as np
import jax
import jax.numpy as jnp
from jax.experimental import pallas as pl
from jax.experimental.pallas import tpu as pltpu

NN = 10000
NP = 10240
BN = 1024
BK = 1024
H = 8
HC = 8
DV = 8
HID = 64
XW = HID + H * HC * DV
KHOP = 3
CST = 1e-5
NUM_TIMESTEPS = 128.0
RESCALE = 4000.0
BE = 8192


def _routing_consts():
    R = np.zeros((64, 512), np.float32)
    T = np.zeros((64, 512), np.float32)
    S = np.zeros((512, 64), np.float32)
    CH = np.zeros((64, 64), np.float32)
    for h in range(H):
        for i in range(HC):
            for j in range(DV):
                c = h * 64 + i * 8 + j
                R[h * 8 + i, c] = 1.0
                T[h * 8 + j, c] = 1.0
                S[c, h * 8 + j] = 1.0
    for a in range(64):
        for b in range(64):
            if a // 8 == b // 8:
                CH[a, b] = 1.0
    return (jnp.asarray(R), jnp.asarray(T), jnp.asarray(S), jnp.asarray(CH))


def _k1(x_ref, ts_ref, Win_ref, bin_ref, Wq_ref, bq_ref, Wk_ref, bk_ref,
        Wv_ref, bv_ref, Wt1_ref, bt1_ref, Wt2_ref, bt2_ref, R_ref, T_ref,
        hw0_ref, X0_ref, Q_ref, hid_ref):
    t = ts_ref[0, 0] / NUM_TIMESTEPS * RESCALE
    c = jax.lax.broadcasted_iota(jnp.int32, (8, HID), 1).astype(jnp.float32)
    fidx = jnp.where(c < 32.0, c, c - 32.0)
    freqs = jnp.exp(fidx * (-np.log(10000.0) / 31.0))
    ang = t * freqs
    temb0 = jnp.where(c < 32.0, jnp.sin(ang), jnp.cos(ang))
    a1 = temb0 @ Wt1_ref[...] + bt1_ref[...]
    a1 = a1 * jax.nn.sigmoid(a1)
    temb = (a1 @ Wt2_ref[...] + bt2_ref[...])[0:1]
    h = jnp.maximum(x_ref[...] @ Win_ref[...] + bin_ref[...] + temb, 0.0)

    def elu1p(z):
        return jnp.where(z > 0, 1.0 + z, jnp.exp(jnp.minimum(z, 0.0)))

    q = elu1p(h @ Wq_ref[...] + bq_ref[...])
    kf = elu1p(h @ Wk_ref[...] + bk_ref[...])
    v = h @ Wv_ref[...] + bv_ref[...]
    m = (kf @ R_ref[...]) * (v @ T_ref[...])
    X0_ref[:, 0:HID] = kf
    X0_ref[:, HID:XW] = m
    Q_ref[...] = q
    hid_ref[...] = v * hw0_ref[0, 0]


def _hop(P_ref, X_ref, Q_ref, hin_ref, R_ref, S_ref, CH_ref, gam_ref,
         Xn_ref, hout_ref, *, nk):
    k = pl.program_id(1)
    part = jnp.dot(P_ref[...], X_ref[...], preferred_element_type=jnp.float32)

    @pl.when(k == 0)
    def _():
        Xn_ref[...] = part

    @pl.when(k != 0)
    def _():
        Xn_ref[...] += part

    @pl.when(k == nk - 1)
    def _():
        xn = Xn_ref[...]
        kf = xn[:, 0:HID]
        m = xn[:, HID:XW]
        q = Q_ref[...]
        qm = (q @ R_ref[...]) * m
        hm = qm @ S_ref[...]
        denom = (q * kf) @ CH_ref[...] + CST
        hout_ref[...] = hin_ref[...] + gam_ref[...] * (hm / denom)


def _kout(hid_ref, Wo_ref, bo_ref, out_ref):
    out_ref[...] = hid_ref[...] @ Wo_ref[...] + bo_ref[...]


def _khead(e_ref, W1_ref, b1_ref, W2_ref, b2_ref, out_ref):
    a = e_ref[...] @ W1_ref[...] + b1_ref[...]
    a = a * jax.nn.sigmoid(a)
    out_ref[...] = a @ W2_ref[...] + b2_ref[...]


def _full(shape):
    return pl.BlockSpec(shape, lambda *_: tuple(0 for _ in shape))


def kernel(x, time_steps, edge_index, full_edge_index, W_in, b_in, Wq, bq,
           Wk, bk, Wv, bv, Wt1, bt1, Wt2, bt2, W_out, b_out, Wf1, bf1,
           Wf2, bf2, hopwise, headwise):
    E = edge_index.shape[1]
    row, col = edge_index[0], edge_index[1]
    deg = jnp.zeros((NN,), jnp.float32).at[col].add(1.0)
    deg_inv = jnp.where(deg > 0, 1.0 / deg, 0.0)
    P = jnp.zeros((NP, NP), jnp.float32).at[col, row].add(deg_inv[row])

    xp = jnp.zeros((NP, x.shape[1]), jnp.float32).at[:NN].set(x)
    R, T, S, CH = _routing_consts()
    lw = jax.nn.softmax(headwise, axis=0)
    gammas = [jnp.repeat(hopwise[h + 1] * lw[:, h], DV).reshape(1, HID)
              for h in range(KHOP)]
    ts = time_steps.reshape(1, 1)
    r1 = lambda b: b.reshape(1, -1)

    ni = NP // BN
    X0, Q, hid = pl.pallas_call(
        _k1,
        grid=(ni,),
        in_specs=[
            pl.BlockSpec((BN, x.shape[1]), lambda i: (i, 0)),
            _full((1, 1)),
            _full(W_in.shape), _full((1, HID)),
            _full(Wq.shape), _full((1, HID)),
            _full(Wk.shape), _full((1, HID)),
            _full(Wv.shape), _full((1, HID)),
            _full(Wt1.shape), _full((1, 4 * HID)),
            _full(Wt2.shape), _full((1, HID)),
            _full(R.shape), _full(T.shape),
            _full((1, 1)),
        ],
        out_specs=[
            pl.BlockSpec((BN, XW), lambda i: (i, 0)),
            pl.BlockSpec((BN, HID), lambda i: (i, 0)),
            pl.BlockSpec((BN, HID), lambda i: (i, 0)),
        ],
        out_shape=[
            jax.ShapeDtypeStruct((NP, XW), jnp.float32),
            jax.ShapeDtypeStruct((NP, HID), jnp.float32),
            jax.ShapeDtypeStruct((NP, HID), jnp.float32),
        ],
    )(xp, ts, W_in, r1(b_in), Wq, r1(bq), Wk, r1(bk), Wv, r1(bv),
      Wt1, r1(bt1), Wt2, r1(bt2), R, T, hopwise[0].reshape(1, 1))

    nk = NP // BK
    import functools
    hop_fn = functools.partial(_hop, nk=nk)
    Pb = P.astype(jnp.bfloat16)
    X = X0
    for h in range(KHOP):
        Xb = X.astype(jnp.bfloat16)
        X, hid = pl.pallas_call(
            hop_fn,
            grid=(ni, nk),
            in_specs=[
                pl.BlockSpec((BN, BK), lambda i, k: (i, k)),
                pl.BlockSpec((BK, XW), lambda i, k: (k, 0)),
                pl.BlockSpec((BN, HID), lambda i, k: (i, 0)),
                pl.BlockSpec((BN, HID), lambda i, k: (i, 0)),
                pl.BlockSpec(R.shape, lambda i, k: (0, 0)),
                pl.BlockSpec(S.shape, lambda i, k: (0, 0)),
                pl.BlockSpec(CH.shape, lambda i, k: (0, 0)),
                pl.BlockSpec((1, HID), lambda i, k: (0, 0)),
            ],
            out_specs=[
                pl.BlockSpec((BN, XW), lambda i, k: (i, 0)),
                pl.BlockSpec((BN, HID), lambda i, k: (i, 0)),
            ],
            out_shape=[
                jax.ShapeDtypeStruct((NP, XW), jnp.float32),
                jax.ShapeDtypeStruct((NP, HID), jnp.float32),
            ],
            compiler_params=pltpu.CompilerParams(
                dimension_semantics=("parallel", "arbitrary")),
        )(Pb, Xb, Q, hid, R, S, CH, gammas[h])

    hid8 = pl.pallas_call(
        _kout,
        grid=(ni,),
        in_specs=[
            pl.BlockSpec((BN, HID), lambda i: (i, 0)),
            _full(W_out.shape), _full((1, DV)),
        ],
        out_specs=pl.BlockSpec((BN, DV), lambda i: (i, 0)),
        out_shape=jax.ShapeDtypeStruct((NP, DV), jnp.float32),
    )(hid, W_out, r1(b_out))[:NN]

    src, dst = full_edge_index[0], full_edge_index[1]
    e_in = jnp.concatenate([hid8[src], hid8[dst]], axis=-1)
    EP = ((E + BE - 1) // BE) * BE
    e_pad = jnp.zeros((EP, 2 * DV), jnp.float32).at[:E].set(e_in)
    logits = pl.pallas_call(
        _khead,
        grid=(EP // BE,),
        in_specs=[
            pl.BlockSpec((BE, 2 * DV), lambda i: (i, 0)),
            _full(Wf1.shape), _full((1, DV)),
            _full(Wf2.shape), _full((1, 1)),
        ],
        out_specs=pl.BlockSpec((BE, 1), lambda i: (i, 0)),
        out_shape=jax.ShapeDtypeStruct((EP, 1), jnp.float32),
    )(e_pad, Wf1, r1(bf1), Wf2, r1(bf2))[:E]

    return (logits, hid8)


# speedup vs baseline: 1.0071x; 1.0071x over previous
import numpy as np
import jax
import jax.numpy as jnp
from jax.experimental import pallas as pl
from jax.experimental.pallas import tpu as pltpu

NN = 10000
NP = 10240
BN = 1024
BK = 1024
H = 8
HC = 8
DV = 8
HID = 64
XW = HID + H * HC * DV
KHOP = 3
CST = 1e-5
NUM_TIMESTEPS = 128.0
RESCALE = 4000.0
BE = 8192


def _routing_consts():
    R = np.zeros((64, 512), np.float32)
    T = np.zeros((64, 512), np.float32)
    S = np.zeros((512, 64), np.float32)
    CH = np.zeros((64, 64), np.float32)
    for h in range(H):
        for i in range(HC):
            for j in range(DV):
                c = h * 64 + i * 8 + j
                R[h * 8 + i, c] = 1.0
                T[h * 8 + j, c] = 1.0
                S[c, h * 8 + j] = 1.0
    for a in range(64):
        for b in range(64):
            if a // 8 == b // 8:
                CH[a, b] = 1.0
    return (jnp.asarray(R), jnp.asarray(T), jnp.asarray(S), jnp.asarray(CH))


def _k1(x_ref, ts_ref, Win_ref, bin_ref, Wq_ref, bq_ref, Wk_ref, bk_ref,
        Wv_ref, bv_ref, Wt1_ref, bt1_ref, Wt2_ref, bt2_ref, R_ref, T_ref,
        hw0_ref, X0_ref, Q_ref, hid_ref):
    t = ts_ref[0, 0] / NUM_TIMESTEPS * RESCALE
    c = jax.lax.broadcasted_iota(jnp.int32, (8, HID), 1).astype(jnp.float32)
    fidx = jnp.where(c < 32.0, c, c - 32.0)
    freqs = jnp.exp(fidx * (-np.log(10000.0) / 31.0))
    ang = t * freqs
    temb0 = jnp.where(c < 32.0, jnp.sin(ang), jnp.cos(ang))
    a1 = temb0 @ Wt1_ref[...] + bt1_ref[...]
    a1 = a1 * jax.nn.sigmoid(a1)
    temb = (a1 @ Wt2_ref[...] + bt2_ref[...])[0:1]
    h = jnp.maximum(x_ref[...] @ Win_ref[...] + bin_ref[...] + temb, 0.0)

    def elu1p(z):
        return jnp.where(z > 0, 1.0 + z, jnp.exp(jnp.minimum(z, 0.0)))

    q = elu1p(h @ Wq_ref[...] + bq_ref[...])
    kf = elu1p(h @ Wk_ref[...] + bk_ref[...])
    v = h @ Wv_ref[...] + bv_ref[...]
    m = (kf @ R_ref[...]) * (v @ T_ref[...])
    X0_ref[:, 0:HID] = kf
    X0_ref[:, HID:XW] = m
    Q_ref[...] = q
    hid_ref[...] = v * hw0_ref[0, 0]


def _hop(P_ref, X_ref, Q_ref, hin_ref, R_ref, S_ref, CH_ref, gam_ref,
         Xn_ref, hout_ref, *, nk):
    k = pl.program_id(1)
    part = jnp.dot(P_ref[...], X_ref[...], preferred_element_type=jnp.float32)

    @pl.when(k == 0)
    def _():
        Xn_ref[...] = part

    @pl.when(k != 0)
    def _():
        Xn_ref[...] += part

    @pl.when(k == nk - 1)
    def _():
        xn = Xn_ref[...]
        kf = xn[:, 0:HID]
        m = xn[:, HID:XW]
        q = Q_ref[...]
        qm = (q @ R_ref[...]) * m
        hm = qm @ S_ref[...]
        denom = (q * kf) @ CH_ref[...] + CST
        hout_ref[...] = hin_ref[...] + gam_ref[...] * (hm / denom)


def _kout(hid_ref, Wo_ref, bo_ref, out_ref):
    out_ref[...] = hid_ref[...] @ Wo_ref[...] + bo_ref[...]


def _khead(e_ref, W1_ref, b1_ref, W2_ref, b2_ref, out_ref):
    a = e_ref[...] @ W1_ref[...] + b1_ref[...]
    a = a * jax.nn.sigmoid(a)
    out_ref[...] = a @ W2_ref[...] + b2_ref[...]


def _full(shape):
    return pl.BlockSpec(shape, lambda *_: tuple(0 for _ in shape))


def kernel(x, time_steps, edge_index, full_edge_index, W_in, b_in, Wq, bq,
           Wk, bk, Wv, bv, Wt1, bt1, Wt2, bt2, W_out, b_out, Wf1, bf1,
           Wf2, bf2, hopwise, headwise):
    E = edge_index.shape[1]
    row, col = edge_index[0], edge_index[1]
    deg = jnp.zeros((NN,), jnp.float32).at[col].add(1.0)
    deg_inv = jnp.where(deg > 0, 1.0 / deg, 0.0)
    P = jnp.zeros((NP, NP), jnp.float32).at[col, row].add(deg_inv[row])

    xp = jnp.zeros((NP, x.shape[1]), jnp.float32).at[:NN].set(x)
    R, T, S, CH = _routing_consts()
    lw = jax.nn.softmax(headwise, axis=0)
    gammas = [jnp.repeat(hopwise[h + 1] * lw[:, h], DV).reshape(1, HID)
              for h in range(KHOP)]
    ts = time_steps.reshape(1, 1)
    r1 = lambda b: b.reshape(1, -1)

    ni = NP // BN
    X0, Q, hid = pl.pallas_call(
        _k1,
        grid=(ni,),
        in_specs=[
            pl.BlockSpec((BN, x.shape[1]), lambda i: (i, 0)),
            _full((1, 1)),
            _full(W_in.shape), _full((1, HID)),
            _full(Wq.shape), _full((1, HID)),
            _full(Wk.shape), _full((1, HID)),
            _full(Wv.shape), _full((1, HID)),
            _full(Wt1.shape), _full((1, 4 * HID)),
            _full(Wt2.shape), _full((1, HID)),
            _full(R.shape), _full(T.shape),
            _full((1, 1)),
        ],
        out_specs=[
            pl.BlockSpec((BN, XW), lambda i: (i, 0)),
            pl.BlockSpec((BN, HID), lambda i: (i, 0)),
            pl.BlockSpec((BN, HID), lambda i: (i, 0)),
        ],
        out_shape=[
            jax.ShapeDtypeStruct((NP, XW), jnp.float32),
            jax.ShapeDtypeStruct((NP, HID), jnp.float32),
            jax.ShapeDtypeStruct((NP, HID), jnp.float32),
        ],
    )(xp, ts, W_in, r1(b_in), Wq, r1(bq), Wk, r1(bk), Wv, r1(bv),
      Wt1, r1(bt1), Wt2, r1(bt2), R, T, hopwise[0].reshape(1, 1))

    nk = NP // BK
    import functools
    hop_fn = functools.partial(_hop, nk=nk)
    X = X0
    for h in range(KHOP):
        X, hid = pl.pallas_call(
            hop_fn,
            grid=(ni, nk),
            in_specs=[
                pl.BlockSpec((BN, BK), lambda i, k: (i, k)),
                pl.BlockSpec((BK, XW), lambda i, k: (k, 0)),
                pl.BlockSpec((BN, HID), lambda i, k: (i, 0)),
                pl.BlockSpec((BN, HID), lambda i, k: (i, 0)),
                pl.BlockSpec(R.shape, lambda i, k: (0, 0)),
                pl.BlockSpec(S.shape, lambda i, k: (0, 0)),
                pl.BlockSpec(CH.shape, lambda i, k: (0, 0)),
                pl.BlockSpec((1, HID), lambda i, k: (0, 0)),
            ],
            out_specs=[
                pl.BlockSpec((BN, XW), lambda i, k: (i, 0)),
                pl.BlockSpec((BN, HID), lambda i, k: (i, 0)),
            ],
            out_shape=[
                jax.ShapeDtypeStruct((NP, XW), jnp.float32),
                jax.ShapeDtypeStruct((NP, HID), jnp.float32),
            ],
            compiler_params=pltpu.CompilerParams(
                dimension_semantics=("parallel", "arbitrary")),
        )(P, X, Q, hid, R, S, CH, gammas[h])

    hid8 = pl.pallas_call(
        _kout,
        grid=(ni,),
        in_specs=[
            pl.BlockSpec((BN, HID), lambda i: (i, 0)),
            _full(W_out.shape), _full((1, DV)),
        ],
        out_specs=pl.BlockSpec((BN, DV), lambda i: (i, 0)),
        out_shape=jax.ShapeDtypeStruct((NP, DV), jnp.float32),
    )(hid, W_out, r1(b_out))[:NN]

    src, dst = full_edge_index[0], full_edge_index[1]
    e_in = jnp.concatenate([hid8[src], hid8[dst]], axis=-1)
    EP = ((E + BE - 1) // BE) * BE
    e_pad = jnp.zeros((EP, 2 * DV), jnp.float32).at[:E].set(e_in)
    logits = pl.pallas_call(
        _khead,
        grid=(EP // BE,),
        in_specs=[
            pl.BlockSpec((BE, 2 * DV), lambda i: (i, 0)),
            _full(Wf1.shape), _full((1, DV)),
            _full(Wf2.shape), _full((1, 1)),
        ],
        out_specs=pl.BlockSpec((BE, 1), lambda i: (i, 0)),
        out_shape=jax.ShapeDtypeStruct((EP, 1), jnp.float32),
    )(e_pad, Wf1, r1(bf1), Wf2, r1(bf2))[:E]

    return (logits, hid8)
